# baseline (speedup 1.0000x reference)
import jax
import jax.numpy as jnp
from jax import lax
from jax.experimental import pallas as pl
from jax.experimental.pallas import tpu as pltpu
from jax.experimental.pallas import tpu_sc as plsc

N = 50000
E = 200000
D = 128
H = 4
C = 32
NRG = 8704
NP = 51200
PAD_DST = 6 * NRG
DUMMY = NP - 16
Q = 14336
E_PAD = 16 * Q
QROWS = Q // 128
SROWS = 4
NSTAGE = QROWS // SROWS
PEND = 640
K = 64
ZB = 32
NBLK = NRG // ZB
NDR = NRG // 32
TDR = NDR // 16
F32 = jnp.float32
I32 = jnp.int32


TCB = 2560


def _proj_kernel(x_ref, w_ref, b_ref, *o_refs):
    z = jnp.dot(x_ref[...], w_ref[...], preferred_element_type=F32) + b_ref[...]
    for k in range(6):
        o_refs[k][...] = z[:, k * D:(k + 1) * D]


def _project(xp, Wcat, bcat):
    return pl.pallas_call(
        _proj_kernel,
        grid=(NP // TCB,),
        in_specs=[
            pl.BlockSpec((TCB, D), lambda i: (i, 0)),
            pl.BlockSpec((D, 6 * D), lambda i: (0, 0)),
            pl.BlockSpec((6 * D,), lambda i: (0,)),
        ],
        out_specs=[pl.BlockSpec((TCB, D), lambda i: (i, 0))] * 6,
        out_shape=[jax.ShapeDtypeStruct((NP, D), F32)] * 6,
    )(xp, Wcat, bcat)


def _normalize_block(af_refs, ad_refs, bias_ref):
    bias = bias_ref[...]
    dsum = [ad_refs[r][...] for r in range(3)]
    parts = []
    for h in range(H):
        s = 0.0
        for r in range(3):
            af = af_refs[r][...]
            den = dsum[r][:, h:h + 1] + 1e-16
            s = s + af[:, h * C:(h + 1) * C] / den
        parts.append(s)
    hcat = jnp.concatenate(parts, axis=1)
    return hcat + jnp.sum(bias, axis=0)[None, :]


def _mid_kernel(f0, f1, f2, d0, d1, d2, bias_ref, w_ref, b_ref, *o_refs):
    hv = _normalize_block((f0, f1, f2), (d0, d1, d2), bias_ref)
    z = jnp.dot(hv, w_ref[...], preferred_element_type=F32) + b_ref[...]
    for k in range(6):
        o_refs[k][...] = z[:, k * D:(k + 1) * D]


def _mid(fs, ds, bias, Wcat, bcat):
    return pl.pallas_call(
        _mid_kernel,
        grid=(NP // TCB,),
        in_specs=[pl.BlockSpec((TCB, D), lambda i: (i, 0))] * 3
        + [pl.BlockSpec((TCB, 4), lambda i: (i, 0))] * 3
        + [
            pl.BlockSpec((3, D), lambda i: (0, 0)),
            pl.BlockSpec((D, 6 * D), lambda i: (0, 0)),
            pl.BlockSpec((6 * D,), lambda i: (0,)),
        ],
        out_specs=[pl.BlockSpec((TCB, D), lambda i: (i, 0))] * 6,
        out_shape=[jax.ShapeDtypeStruct((NP, D), F32)] * 6,
    )(fs[0], fs[1], fs[2], ds[0], ds[1], ds[2], bias, Wcat, bcat)


FINB = 2048


def _fin_kernel(f0, f1, f2, d0, d1, d2, bias_ref, w_ref, b_ref, o_ref):
    hv = _normalize_block((f0, f1, f2), (d0, d1, d2), bias_ref)
    hm = jax.nn.relu(hv.reshape(FINB // 4, 4 * D))
    o_ref[...] = (
        jnp.dot(hm, w_ref[...], preferred_element_type=F32) + b_ref[...]
    )[None]


def _final(fs, ds, bias, Wfc, bfc):
    out = pl.pallas_call(
        _fin_kernel,
        grid=(NP // FINB,),
        in_specs=[pl.BlockSpec((FINB, D), lambda i: (i, 0))] * 3
        + [pl.BlockSpec((FINB, 4), lambda i: (i, 0))] * 3
        + [
            pl.BlockSpec((3, D), lambda i: (0, 0)),
            pl.BlockSpec((4 * D, 4), lambda i: (0, 0)),
            pl.BlockSpec((4,), lambda i: (0,)),
        ],
        out_specs=pl.BlockSpec((1, FINB // 4, 4), lambda i: (i, 0, 0)),
        out_shape=jax.ShapeDtypeStruct((NP // FINB, FINB // 4, 4), F32),
    )(fs[0], fs[1], fs[2], ds[0], ds[1], ds[2], bias, Wfc, bfc)
    return out.reshape(NP // 4, 4)[:N // 4]


def _sc_body(s0, d0, s1, d1, s2, d2, xl0, xl1, xl2, xr0, xr1, xr2, att_hbm,
             of0, of1, of2, od0, od1, od2,
             acc, accd, src_st, dst_st, rows_idx, p_src, p_dst, xl_b, xr_b,
             y_b, yd_b, fbuf, idx_loc, drow, frows, frd, att_r,
             sem1, sem2):
    core = lax.axis_index("c")
    sub = lax.axis_index("s")
    iota = lax.iota(I32, 16)
    fz = jnp.zeros((16,), F32)
    iz = jnp.zeros((16,), I32)

    def _z1(r, _):
        for u in range(D // 16):
            yd_b[r, pl.ds(u * 16, 16)] = fz
        return 0
    lax.fori_loop(0, K, _z1, 0)

    srcs = (s0, s1, s2)
    dsts = (d0, d1, d2)
    xls = (xl0, xl1, xl2)
    xrs = (xr0, xr1, xr2)
    ofs = (of0, of1, of2)
    ods = (od0, od1, od2)

    for rel in range(3):
        plsc.store_scatter(rows_idx, [iota], rel * 8 + iota, mask=iota < SROWS)
        pltpu.async_copy(att_hbm.at[rows_idx], att_r, sem1).wait()

        def _range_body(rg, _, rel=rel):
            lo = (core * 3 + rg) * NRG
            hi = lo + NRG

            def _zf(r, _):
                for u in range(D // 16):
                    fbuf[r, pl.ds(u * 16, 16)] = fz
                return 0
            lax.fori_loop(0, ZB, _zf, 0)

            def _zero(j, _):
                b = sub + j * 16
                pltpu.sync_copy(fbuf, acc.at[pl.ds(b * ZB, ZB)])
                return 0
            lax.fori_loop(0, NBLK // 16, _zero, 0)

            pltpu.sync_copy(fbuf.at[pl.ds(0, TDR)],
                            accd.at[pl.ds(sub * TDR, TDR)])
            plsc.subcore_barrier()

            def _stage(sidx, cnt, rel=rel):
                plsc.store_scatter(rows_idx, [iota],
                                   sub * QROWS + sidx * SROWS + iota,
                                   mask=iota < SROWS)
                gs = pltpu.async_copy(srcs[rel].at[rows_idx], src_st, sem1)
                gd = pltpu.async_copy(dsts[rel].at[rows_idx], dst_st, sem2)
                gs.wait()
                gd.wait()

                def _scan(i, cnt):
                    row = i >> 3
                    u = i & 7
                    sv = plsc.bitcast(src_st[row, pl.ds(u * 16, 16)], I32)
                    dv = plsc.bitcast(dst_st[row, pl.ds(u * 16, 16)], I32)
                    m = (dv >= lo) & (dv < hi)
                    cs = plsc.cumsum(m.astype(I32))
                    pos = cnt + cs - 1
                    plsc.store_scatter(p_src, [pos], sv, mask=m)
                    plsc.store_scatter(p_dst, [pos], dv, mask=m)
                    return cnt + jnp.max(cs)
                cnt = lax.fori_loop(0, SROWS * 8, _scan, cnt)

                @pl.when(sidx == NSTAGE - 1)
                def _():
                    for u in range(K // 16):
                        p_src[pl.ds(cnt + u * 16, 16)] = iz
                        p_dst[pl.ds(cnt + u * 16, 16)] = iz + DUMMY

                nfull = jnp.where(sidx == NSTAGE - 1,
                                  (cnt + K - 1) // K, cnt // K)

                def _chunk(j, _, rel=rel):
                    off = j * K
                    gl = pltpu.async_copy(
                        xls[rel].at[p_src.at[pl.ds(off, K)]], xl_b, sem1)
                    gr = pltpu.async_copy(
                        xrs[rel].at[p_dst.at[pl.ds(off, K)]], xr_b, sem2)
                    gl.wait()
                    gr.wait()

                    def _group(g, _):
                        rows = iota + g * 16
                        dv = p_dst[pl.ds(off + g * 16, 16)]
                        loc = jnp.minimum(dv - lo, NRG - 1)
                        vmask = (dv != DUMMY).astype(F32)
                        idx_loc[pl.ds(g * 16, 16)] = loc
                        ex = []
                        for h in range(H):
                            def _dot(ci, lg, h=h):
                                cc = jnp.full((16,), h * C, I32) + ci
                                a = plsc.load_gather(xl_b, [rows, cc])
                                b = plsc.load_gather(xr_b, [rows, cc])
                                s = a + b
                                ev = (jnp.maximum(s, 0.0)
                                      + 0.2 * jnp.minimum(s, 0.0))
                                ac = plsc.load_gather(att_r, [iz, cc])
                                return lg + ev * ac
                            lg = lax.fori_loop(0, C, _dot, fz)
                            ex.append(jnp.exp(lg) * vmask)

                        def _ystore(c, _):
                            cc = jnp.full((16,), 0, I32) + c
                            a = plsc.load_gather(xl_b, [rows, cc])
                            es = jnp.where(c < 2 * C,
                                           jnp.where(c < C, ex[0], ex[1]),
                                           jnp.where(c < 3 * C, ex[2], ex[3]))
                            plsc.store_scatter(y_b, [rows, cc], a * es)
                            return 0
                        lax.fori_loop(0, D, _ystore, 0)
                        drow[pl.ds(g * 16, 16)] = loc >> 5
                        dcol = (loc & 31) * 4
                        for h in range(H):
                            plsc.store_scatter(yd_b, [rows, dcol + h], ex[h])
                        return 0
                    lax.fori_loop(0, K // 16, _group, 0)
                    pltpu.sync_copy(y_b, acc.at[idx_loc], add=True)
                    pltpu.sync_copy(yd_b, accd.at[drow], add=True)

                    def _rz(g, _):
                        rows = iota + g * 16
                        dv = p_dst[pl.ds(off + g * 16, 16)]
                        loc = jnp.minimum(dv - lo, NRG - 1)
                        dcol = (loc & 31) * 4
                        for h in range(H):
                            plsc.store_scatter(yd_b, [rows, dcol + h], fz)
                        return 0
                    lax.fori_loop(0, K // 16, _rz, 0)
                    return 0
                lax.fori_loop(0, nfull, _chunk, 0)

                base = nfull * K
                for u in range(K // 16):
                    sv = p_src[pl.ds(base + u * 16, 16)]
                    dv = p_dst[pl.ds(base + u * 16, 16)]
                    p_src[pl.ds(u * 16, 16)] = sv
                    p_dst[pl.ds(u * 16, 16)] = dv
                return cnt - base
            lax.fori_loop(0, NSTAGE, _stage, jnp.int32(0))
            plsc.subcore_barrier()

            def _flushf(j, _, rel=rel):
                b = sub + j * 16
                for u in range(ZB // 16):
                    frows[pl.ds(u * 16, 16)] = lo + b * ZB + u * 16 + iota
                pltpu.sync_copy(acc.at[pl.ds(b * ZB, ZB)], fbuf)
                pltpu.async_copy(fbuf, ofs[rel].at[frows], sem1).wait()
                return 0
            lax.fori_loop(0, NBLK // 16, _flushf, 0)

            gbase = (core * 3 + rg) * NDR + sub * TDR
            frd[pl.ds(0, 16)] = gbase + iota
            plsc.store_scatter(frd, [iota + 16], gbase + 16 + iota,
                               mask=iota < TDR - 16)
            pltpu.sync_copy(accd.at[pl.ds(sub * TDR, TDR)],
                            fbuf.at[pl.ds(0, TDR)])
            pltpu.async_copy(fbuf.at[pl.ds(0, TDR)], ods[rel].at[frd],
                             sem2).wait()
            plsc.subcore_barrier()
            return 0

        lax.fori_loop(0, 3, _range_body, 0)


def _sc_edges(srcs, dsts, xls, xrs, att):
    mesh = plsc.VectorSubcoreMesh(core_axis_name="c", subcore_axis_name="s")
    f = pl.kernel(
        _sc_body,
        out_type=[jax.ShapeDtypeStruct((NP, D), F32)] * 3
        + [jax.ShapeDtypeStruct((6 * NDR, D), F32)] * 3,
        compiler_params=pltpu.CompilerParams(needs_layout_passes=False),
        mesh=mesh,
        scratch_types=[
            pltpu.VMEM_SHARED((NRG, D), F32),
            pltpu.VMEM_SHARED((NDR, D), F32),
            pltpu.VMEM((SROWS, 128), F32),
            pltpu.VMEM((SROWS, 128), F32),
            pltpu.VMEM((SROWS,), I32),
            pltpu.VMEM((PEND,), I32),
            pltpu.VMEM((PEND,), I32),
            pltpu.VMEM((K, D), F32),
            pltpu.VMEM((K, D), F32),
            pltpu.VMEM((K, D), F32),
            pltpu.VMEM((K, D), F32),
            pltpu.VMEM((ZB, D), F32),
            pltpu.VMEM((K,), I32),
            pltpu.VMEM((K,), I32),
            pltpu.VMEM((ZB,), I32),
            pltpu.VMEM((TDR,), I32),
            pltpu.VMEM((SROWS, D), F32),
            pltpu.SemaphoreType.DMA,
            pltpu.SemaphoreType.DMA,
        ],
    )
    return f(srcs[0], dsts[0], srcs[1], dsts[1], srcs[2], dsts[2],
             xls[0], xls[1], xls[2], xrs[0], xrs[1], xrs[2], att)


def _pad_edges(ei):
    src = jnp.concatenate([ei[0], jnp.zeros((E_PAD - E,), I32)])
    dst = jnp.concatenate([ei[1], jnp.full((E_PAD - E,), PAD_DST, I32)])
    return (jax.lax.bitcast_convert_type(src, F32).reshape(16 * QROWS, 128),
            jax.lax.bitcast_convert_type(dst, F32).reshape(16 * QROWS, 128))


def _wcat(Wl, bl, Wr, br):
    Wcat = jnp.concatenate([Wl[0], Wr[0], Wl[1], Wr[1], Wl[2], Wr[2]], axis=1)
    bcat = jnp.concatenate([bl[0], br[0], bl[1], br[1], bl[2], br[2]])
    return Wcat, bcat


def kernel(x, edge_index_for, edge_index_against, edge_index_vote,
           Wl1, bl1, Wr1, br1, att1, bias1,
           Wl2, bl2, Wr2, br2, att2, bias2,
           Wfc, bfc):
    pads = [_pad_edges(e) for e in
            (edge_index_for, edge_index_against, edge_index_vote)]
    srcs = tuple(p[0] for p in pads)
    dsts = tuple(p[1] for p in pads)
    Wc1, bc1 = _wcat(Wl1, bl1, Wr1, br1)
    Wc2, bc2 = _wcat(Wl2, bl2, Wr2, br2)
    xp = jnp.concatenate([x, jnp.zeros((NP - N, D), F32)], axis=0)
    z1 = _project(xp, Wc1, bc1)
    att1f = jnp.zeros((24, D), F32).at[jnp.array([0, 8, 16])].set(att1.reshape(3, D))
    att2f = jnp.zeros((24, D), F32).at[jnp.array([0, 8, 16])].set(att2.reshape(3, D))
    a1 = _sc_edges(srcs, dsts, (z1[0], z1[2], z1[4]), (z1[1], z1[3], z1[5]),
                   att1f)
    d1 = tuple(a.reshape(6 * NDR * 32, 4)[:NP] for a in a1[3:6])
    z2 = _mid(a1[0:3], d1, bias1, Wc2, bc2)
    a2 = _sc_edges(srcs, dsts, (z2[0], z2[2], z2[4]), (z2[1], z2[3], z2[5]),
                   att2f)
    d2 = tuple(a.reshape(6 * NDR * 32, 4)[:NP] for a in a2[3:6])
    return _final(a2[0:3], d2, bias2, Wfc, bfc)


# speedup vs baseline: 1.0879x; 1.0879x over previous
import jax
import jax.numpy as jnp
from jax import lax
from jax.experimental import pallas as pl
from jax.experimental.pallas import tpu as pltpu
from jax.experimental.pallas import tpu_sc as plsc

N = 50000
E = 200000
D = 128
H = 4
C = 32
NRG = 12800
NP = 51200
PAD_DST = 4 * NRG
DUMMY = NP - 16
Q = 14336
E_PAD = 16 * Q
QROWS = Q // 128
SROWS = 4
NSTAGE = QROWS // SROWS
PEND = 576
K = 32
ZB = 32
NBLK = NRG // ZB
NDR = NRG // 32
TDR = NDR // 16
F32 = jnp.float32
I32 = jnp.int32


TCB = 2560


def _proj_kernel(x_ref, w_ref, b_ref, *o_refs):
    z = jnp.dot(x_ref[...], w_ref[...], preferred_element_type=F32) + b_ref[...]
    for k in range(6):
        o_refs[k][...] = z[:, k * D:(k + 1) * D]


def _project(xp, Wcat, bcat):
    return pl.pallas_call(
        _proj_kernel,
        grid=(NP // TCB,),
        in_specs=[
            pl.BlockSpec((TCB, D), lambda i: (i, 0)),
            pl.BlockSpec((D, 6 * D), lambda i: (0, 0)),
            pl.BlockSpec((6 * D,), lambda i: (0,)),
        ],
        out_specs=[pl.BlockSpec((TCB, D), lambda i: (i, 0))] * 6,
        out_shape=[jax.ShapeDtypeStruct((NP, D), F32)] * 6,
    )(xp, Wcat, bcat)


def _normalize_block(af_refs, ad_refs, bias_ref):
    bias = bias_ref[...]
    dsum = [ad_refs[r][...] for r in range(3)]
    parts = []
    for h in range(H):
        s = 0.0
        for r in range(3):
            af = af_refs[r][...]
            den = dsum[r][:, h:h + 1] + 1e-16
            s = s + af[:, h * C:(h + 1) * C] / den
        parts.append(s)
    hcat = jnp.concatenate(parts, axis=1)
    return hcat + jnp.sum(bias, axis=0)[None, :]


def _mid_kernel(f0, f1, f2, d0, d1, d2, bias_ref, w_ref, b_ref, *o_refs):
    hv = _normalize_block((f0, f1, f2), (d0, d1, d2), bias_ref)
    z = jnp.dot(hv, w_ref[...], preferred_element_type=F32) + b_ref[...]
    for k in range(6):
        o_refs[k][...] = z[:, k * D:(k + 1) * D]


def _mid(fs, ds, bias, Wcat, bcat):
    return pl.pallas_call(
        _mid_kernel,
        grid=(NP // TCB,),
        in_specs=[pl.BlockSpec((TCB, D), lambda i: (i, 0))] * 3
        + [pl.BlockSpec((TCB, 4), lambda i: (i, 0))] * 3
        + [
            pl.BlockSpec((3, D), lambda i: (0, 0)),
            pl.BlockSpec((D, 6 * D), lambda i: (0, 0)),
            pl.BlockSpec((6 * D,), lambda i: (0,)),
        ],
        out_specs=[pl.BlockSpec((TCB, D), lambda i: (i, 0))] * 6,
        out_shape=[jax.ShapeDtypeStruct((NP, D), F32)] * 6,
    )(fs[0], fs[1], fs[2], ds[0], ds[1], ds[2], bias, Wcat, bcat)


FINB = 2048


def _fin_kernel(f0, f1, f2, d0, d1, d2, bias_ref, w_ref, b_ref, o_ref):
    hv = _normalize_block((f0, f1, f2), (d0, d1, d2), bias_ref)
    hm = jax.nn.relu(hv.reshape(FINB // 4, 4 * D))
    o_ref[...] = (
        jnp.dot(hm, w_ref[...], preferred_element_type=F32) + b_ref[...]
    )[None]


def _final(fs, ds, bias, Wfc, bfc):
    out = pl.pallas_call(
        _fin_kernel,
        grid=(NP // FINB,),
        in_specs=[pl.BlockSpec((FINB, D), lambda i: (i, 0))] * 3
        + [pl.BlockSpec((FINB, 4), lambda i: (i, 0))] * 3
        + [
            pl.BlockSpec((3, D), lambda i: (0, 0)),
            pl.BlockSpec((4 * D, 4), lambda i: (0, 0)),
            pl.BlockSpec((4,), lambda i: (0,)),
        ],
        out_specs=pl.BlockSpec((1, FINB // 4, 4), lambda i: (i, 0, 0)),
        out_shape=jax.ShapeDtypeStruct((NP // FINB, FINB // 4, 4), F32),
    )(fs[0], fs[1], fs[2], ds[0], ds[1], ds[2], bias, Wfc, bfc)
    return out.reshape(NP // 4, 4)[:N // 4]


def _sc_body(s0, d0, s1, d1, s2, d2, xl0, xl1, xl2, xr0, xr1, xr2, att_hbm,
             of0, of1, of2, od0, od1, od2,
             acc, accd, src_st, dst_st, rows_idx, p_src, p_dst, xl_b, xr_b,
             y_b, yd_b, fbuf, idx_loc, drow, frows, frd, att_r,
             sem1, sem2):
    core = lax.axis_index("c")
    sub = lax.axis_index("s")
    iota = lax.iota(I32, 16)
    fz = jnp.zeros((16,), F32)
    iz = jnp.zeros((16,), I32)

    def _z1(r, _):
        for u in range(D // 16):
            yd_b[r, pl.ds(u * 16, 16)] = fz
        return 0
    lax.fori_loop(0, K, _z1, 0)

    srcs = (s0, s1, s2)
    dsts = (d0, d1, d2)
    xls = (xl0, xl1, xl2)
    xrs = (xr0, xr1, xr2)
    ofs = (of0, of1, of2)
    ods = (od0, od1, od2)

    for rel in range(3):
        plsc.store_scatter(rows_idx, [iota], rel * 8 + iota, mask=iota < SROWS)
        pltpu.async_copy(att_hbm.at[rows_idx], att_r, sem1).wait()

        def _range_body(rg, _, rel=rel):
            lo = (core * 2 + rg) * NRG
            hi = lo + NRG

            def _zf(r, _):
                for u in range(D // 16):
                    fbuf[r, pl.ds(u * 16, 16)] = fz
                return 0
            lax.fori_loop(0, ZB, _zf, 0)

            def _zero(j, _):
                b = sub + j * 16
                pltpu.sync_copy(fbuf, acc.at[pl.ds(b * ZB, ZB)])
                return 0
            lax.fori_loop(0, NBLK // 16, _zero, 0)

            pltpu.sync_copy(fbuf.at[pl.ds(0, TDR)],
                            accd.at[pl.ds(sub * TDR, TDR)])
            plsc.subcore_barrier()

            def _stage(sidx, cnt, rel=rel):
                plsc.store_scatter(rows_idx, [iota],
                                   sub * QROWS + sidx * SROWS + iota,
                                   mask=iota < SROWS)
                gs = pltpu.async_copy(srcs[rel].at[rows_idx], src_st, sem1)
                gd = pltpu.async_copy(dsts[rel].at[rows_idx], dst_st, sem2)
                gs.wait()
                gd.wait()

                def _scan(i, cnt):
                    row = i >> 3
                    u = i & 7
                    sv = plsc.bitcast(src_st[row, pl.ds(u * 16, 16)], I32)
                    dv = plsc.bitcast(dst_st[row, pl.ds(u * 16, 16)], I32)
                    m = (dv >= lo) & (dv < hi)
                    cs = plsc.cumsum(m.astype(I32))
                    pos = cnt + cs - 1
                    plsc.store_scatter(p_src, [pos], sv, mask=m)
                    plsc.store_scatter(p_dst, [pos], dv, mask=m)
                    return cnt + jnp.max(cs)
                cnt = lax.fori_loop(0, SROWS * 8, _scan, cnt)

                @pl.when(sidx == NSTAGE - 1)
                def _():
                    for u in range(K // 16):
                        p_src[pl.ds(cnt + u * 16, 16)] = iz
                        p_dst[pl.ds(cnt + u * 16, 16)] = iz + DUMMY

                nfull = jnp.where(sidx == NSTAGE - 1,
                                  (cnt + K - 1) // K, cnt // K)

                def _chunk(j, _, rel=rel):
                    off = j * K
                    gl = pltpu.async_copy(
                        xls[rel].at[p_src.at[pl.ds(off, K)]], xl_b, sem1)
                    gr = pltpu.async_copy(
                        xrs[rel].at[p_dst.at[pl.ds(off, K)]], xr_b, sem2)
                    gl.wait()
                    gr.wait()

                    def _group(g, _):
                        rows = iota + g * 16
                        dv = p_dst[pl.ds(off + g * 16, 16)]
                        loc = jnp.minimum(dv - lo, NRG - 1)
                        vmask = (dv != DUMMY).astype(F32)
                        idx_loc[pl.ds(g * 16, 16)] = loc
                        ex = []
                        for h in range(H):
                            def _dot(ci, lg, h=h):
                                acc4 = lg
                                for v in range(4):
                                    cc = (jnp.full((16,), h * C, I32)
                                          + ci * 4 + v)
                                    a = plsc.load_gather(xl_b, [rows, cc])
                                    b = plsc.load_gather(xr_b, [rows, cc])
                                    s = a + b
                                    ev = (jnp.maximum(s, 0.0)
                                          + 0.2 * jnp.minimum(s, 0.0))
                                    ac = plsc.load_gather(att_r, [iz, cc])
                                    acc4 = acc4 + ev * ac
                                return acc4
                            lg = lax.fori_loop(0, C // 4, _dot, fz)
                            ex.append(jnp.exp(lg) * vmask)

                        def _ystore(c4, _):
                            for v in range(4):
                                c = c4 * 4 + v
                                cc = jnp.full((16,), 0, I32) + c
                                a = plsc.load_gather(xl_b, [rows, cc])
                                es = jnp.where(
                                    c < 2 * C,
                                    jnp.where(c < C, ex[0], ex[1]),
                                    jnp.where(c < 3 * C, ex[2], ex[3]))
                                plsc.store_scatter(y_b, [rows, cc], a * es)
                            return 0
                        lax.fori_loop(0, D // 4, _ystore, 0)
                        drow[pl.ds(g * 16, 16)] = loc >> 5
                        dcol = (loc & 31) * 4
                        for h in range(H):
                            plsc.store_scatter(yd_b, [rows, dcol + h], ex[h])
                        return 0
                    lax.fori_loop(0, K // 16, _group, 0)
                    pltpu.sync_copy(y_b, acc.at[idx_loc], add=True)
                    pltpu.sync_copy(yd_b, accd.at[drow], add=True)

                    def _rz(g, _):
                        rows = iota + g * 16
                        dv = p_dst[pl.ds(off + g * 16, 16)]
                        loc = jnp.minimum(dv - lo, NRG - 1)
                        dcol = (loc & 31) * 4
                        for h in range(H):
                            plsc.store_scatter(yd_b, [rows, dcol + h], fz)
                        return 0
                    lax.fori_loop(0, K // 16, _rz, 0)
                    return 0
                lax.fori_loop(0, nfull, _chunk, 0)

                base = nfull * K
                for u in range(K // 16):
                    sv = p_src[pl.ds(base + u * 16, 16)]
                    dv = p_dst[pl.ds(base + u * 16, 16)]
                    p_src[pl.ds(u * 16, 16)] = sv
                    p_dst[pl.ds(u * 16, 16)] = dv
                return cnt - base
            lax.fori_loop(0, NSTAGE, _stage, jnp.int32(0))
            plsc.subcore_barrier()

            def _flushf(j, _, rel=rel):
                b = sub + j * 16
                for u in range(ZB // 16):
                    frows[pl.ds(u * 16, 16)] = lo + b * ZB + u * 16 + iota
                pltpu.sync_copy(acc.at[pl.ds(b * ZB, ZB)], fbuf)
                pltpu.async_copy(fbuf, ofs[rel].at[frows], sem1).wait()
                return 0
            lax.fori_loop(0, NBLK // 16, _flushf, 0)

            gbase = (core * 2 + rg) * NDR + sub * TDR
            frd[pl.ds(0, 16)] = gbase + iota
            plsc.store_scatter(frd, [iota + 16], gbase + 16 + iota,
                               mask=iota < TDR - 16)
            pltpu.sync_copy(accd.at[pl.ds(sub * TDR, TDR)],
                            fbuf.at[pl.ds(0, TDR)])
            pltpu.async_copy(fbuf.at[pl.ds(0, TDR)], ods[rel].at[frd],
                             sem2).wait()
            plsc.subcore_barrier()
            return 0

        lax.fori_loop(0, 2, _range_body, 0)


def _sc_edges(srcs, dsts, xls, xrs, att):
    mesh = plsc.VectorSubcoreMesh(core_axis_name="c", subcore_axis_name="s")
    f = pl.kernel(
        _sc_body,
        out_type=[jax.ShapeDtypeStruct((NP, D), F32)] * 3
        + [jax.ShapeDtypeStruct((4 * NDR, D), F32)] * 3,
        compiler_params=pltpu.CompilerParams(needs_layout_passes=False),
        mesh=mesh,
        scratch_types=[
            pltpu.VMEM_SHARED((NRG, D), F32),
            pltpu.VMEM_SHARED((NDR, D), F32),
            pltpu.VMEM((SROWS, 128), F32),
            pltpu.VMEM((SROWS, 128), F32),
            pltpu.VMEM((SROWS,), I32),
            pltpu.VMEM((PEND,), I32),
            pltpu.VMEM((PEND,), I32),
            pltpu.VMEM((K, D), F32),
            pltpu.VMEM((K, D), F32),
            pltpu.VMEM((K, D), F32),
            pltpu.VMEM((K, D), F32),
            pltpu.VMEM((ZB, D), F32),
            pltpu.VMEM((K,), I32),
            pltpu.VMEM((K,), I32),
            pltpu.VMEM((ZB,), I32),
            pltpu.VMEM((TDR,), I32),
            pltpu.VMEM((SROWS, D), F32),
            pltpu.SemaphoreType.DMA,
            pltpu.SemaphoreType.DMA,
        ],
    )
    return f(srcs[0], dsts[0], srcs[1], dsts[1], srcs[2], dsts[2],
             xls[0], xls[1], xls[2], xrs[0], xrs[1], xrs[2], att)


def _pad_edges(ei):
    src = jnp.concatenate([ei[0], jnp.zeros((E_PAD - E,), I32)])
    dst = jnp.concatenate([ei[1], jnp.full((E_PAD - E,), PAD_DST, I32)])
    return (jax.lax.bitcast_convert_type(src, F32).reshape(16 * QROWS, 128),
            jax.lax.bitcast_convert_type(dst, F32).reshape(16 * QROWS, 128))


def _wcat(Wl, bl, Wr, br):
    Wcat = jnp.concatenate([Wl[0], Wr[0], Wl[1], Wr[1], Wl[2], Wr[2]], axis=1)
    bcat = jnp.concatenate([bl[0], br[0], bl[1], br[1], bl[2], br[2]])
    return Wcat, bcat


def kernel(x, edge_index_for, edge_index_against, edge_index_vote,
           Wl1, bl1, Wr1, br1, att1, bias1,
           Wl2, bl2, Wr2, br2, att2, bias2,
           Wfc, bfc):
    pads = [_pad_edges(e) for e in
            (edge_index_for, edge_index_against, edge_index_vote)]
    srcs = tuple(p[0] for p in pads)
    dsts = tuple(p[1] for p in pads)
    Wc1, bc1 = _wcat(Wl1, bl1, Wr1, br1)
    Wc2, bc2 = _wcat(Wl2, bl2, Wr2, br2)
    xp = jnp.concatenate([x, jnp.zeros((NP - N, D), F32)], axis=0)
    z1 = _project(xp, Wc1, bc1)
    att1f = jnp.zeros((24, D), F32).at[jnp.array([0, 8, 16])].set(att1.reshape(3, D))
    att2f = jnp.zeros((24, D), F32).at[jnp.array([0, 8, 16])].set(att2.reshape(3, D))
    a1 = _sc_edges(srcs, dsts, (z1[0], z1[2], z1[4]), (z1[1], z1[3], z1[5]),
                   att1f)
    d1 = tuple(a.reshape(4 * NDR * 32, 4)[:NP] for a in a1[3:6])
    z2 = _mid(a1[0:3], d1, bias1, Wc2, bc2)
    a2 = _sc_edges(srcs, dsts, (z2[0], z2[2], z2[4]), (z2[1], z2[3], z2[5]),
                   att2f)
    d2 = tuple(a.reshape(4 * NDR * 32, 4)[:NP] for a in a2[3:6])
    return _final(a2[0:3], d2, bias2, Wfc, bfc)
